# baseline (speedup 1.0000x reference)
import jax
import jax.numpy as jnp
from jax import lax
from jax.experimental import pallas as pl
from jax.experimental.pallas import tpu as pltpu
from jax.experimental.pallas import tpu_sc as plsc

_BATCH = 32768
_NIN = 64
_NOUT = 256
_NC = 2
_NS = 16
_NW = _NC * _NS
_ROWS_PER_W = _BATCH // _NW
_CHUNK = 64
_NCHUNKS = _ROWS_PER_W // _CHUNK
_L = 16
_G = _NIN // _L
_U = 4
_TI = 8
_TC = 128


def _sc_body(q_hbm, w_hbm, idx_hbm, out_hbm,
             x_v, out_v0, out_v1, w_v, idx_v,
             sem_x, sem_o0, sem_o1):
    wid = lax.axis_index("s") * _NC + lax.axis_index("c")
    tj0 = wid * (_ROWS_PER_W // _TC)

    out_bufs = (out_v0, out_v1)
    o_sems = (sem_o0, sem_o1)

    x_dma = pltpu.async_copy(
        q_hbm.at[:, pl.ds(tj0, _ROWS_PER_W // _TC)], x_v, sem_x)

    pltpu.sync_copy(w_hbm, w_v)
    pltpu.sync_copy(idx_hbm, idx_v)

    zero = jnp.zeros((_L,), jnp.float32)

    def zero_body(i, carry):
        r = i // (_NOUT // _L)
        k = (i % (_NOUT // _L)) * _L
        for b in range(2):
            out_bufs[b][r, pl.ds(k, _L)] = zero
            out_bufs[b][r + 1, pl.ds(k, _L)] = zero
        return carry

    lax.fori_loop(0, _CHUNK // 2 * (_NOUT // _L), zero_body, 0,
                  unroll=4)

    w_regs = tuple(w_v[pl.ds(g * _L, _L)] for g in range(_G))
    idx_regs = tuple(idx_v[pl.ds(g * _L, _L)] for g in range(_G))
    lane = lax.iota(jnp.int32, _L)
    ti_regs = tuple(2 * g + lane // _TI for g in range(_G))
    i_regs = tuple(lane % _TI for g in range(_G))

    x_dma.wait()

    o_dmas = [None, None]
    for ci in range(_NCHUNKS):
        b = ci % 2
        row0 = wid * _ROWS_PER_W + ci * _CHUNK
        tjj = ci // (_TC // _CHUNK)
        c0 = (ci % (_TC // _CHUNK)) * _CHUNK
        if o_dmas[b] is not None:
            o_dmas[b].wait()

        out_v = out_bufs[b]

        def row_body(k, carry):
            w_r, idx_r, ti_r, i_r = carry
            for u in range(_U):
                q = k * _U + u
                qsplat = jnp.full((_L,), q, jnp.int32)
                csplat = qsplat + c0
                tsplat = jnp.full((_L,), tjj, jnp.int32)
                vals = tuple(
                    plsc.load_gather(x_v, [ti_r[g], tsplat, i_r[g], csplat])
                    * w_r[g]
                    for g in range(_G))
                for g in range(_G):
                    plsc.store_scatter(out_v, [qsplat, idx_r[g]], vals[g])
            return carry

        lax.fori_loop(0, _CHUNK // _U, row_body,
                      (w_regs, idx_regs, ti_regs, i_regs))

        o_dmas[b] = pltpu.async_copy(
            out_v, out_hbm.at[pl.ds(row0, _CHUNK)], o_sems[b])

    for d in o_dmas:
        if d is not None:
            d.wait()


def kernel(X_in, weights, input_node_order):
    mesh = plsc.VectorSubcoreMesh(
        core_axis_name="c", subcore_axis_name="s",
        num_cores=_NC, num_subcores=_NS,
    )
    f = pl.kernel(
        _sc_body,
        out_type=jax.ShapeDtypeStruct((_BATCH, _NOUT), jnp.float32),
        mesh=mesh,
        compiler_params=pltpu.CompilerParams(needs_layout_passes=False),
        scratch_types=[
            pltpu.VMEM((_TI, _ROWS_PER_W // _TC, _TI, _TC), jnp.float32),
            pltpu.VMEM((_CHUNK, _NOUT), jnp.float32),
            pltpu.VMEM((_CHUNK, _NOUT), jnp.float32),
            pltpu.VMEM((_NIN,), jnp.float32),
            pltpu.VMEM((_NIN,), jnp.int32),
            pltpu.SemaphoreType.DMA,
            pltpu.SemaphoreType.DMA,
            pltpu.SemaphoreType.DMA,
        ],
    )
    q = X_in.reshape(_BATCH // _TC, _TC, _TI, _TI).transpose(2, 0, 3, 1)
    return f(q, weights, input_node_order)
